# Initial kernel scaffold; baseline (speedup 1.0000x reference)
import jax
import jax.numpy as jnp
from jax.experimental import pallas as pl


def kernel(token_ids, weight):
    raise NotImplementedError("write your pallas kernel here")


# speedup vs baseline: 1.0946x; 1.0946x over previous
import functools

import jax
import jax.numpy as jnp
from jax import lax
from jax.experimental import pallas as pl
from jax.experimental.pallas import tpu as pltpu
from jax.experimental.pallas import tpu_sc as plsc

_EMBED_DIM = 32
_NUM_CORES = 2
_NUM_SUBCORES = 16
_NUM_WORKERS = _NUM_CORES * _NUM_SUBCORES
_CHUNK = 1024


def _sc_gather(idx_flat, weight):
    b_total = idx_flat.shape[0]
    d = weight.shape[1]
    b_per_w = b_total // _NUM_WORKERS
    nchunks = b_per_w // _CHUNK
    mesh = plsc.VectorSubcoreMesh(core_axis_name="c", subcore_axis_name="s")

    @functools.partial(
        pl.kernel,
        mesh=mesh,
        out_type=jax.ShapeDtypeStruct((b_total, d), jnp.float32),
        scratch_types=[
            pltpu.VMEM((_CHUNK,), jnp.int32),
            pltpu.VMEM((_CHUNK, d), jnp.float32),
            pltpu.SemaphoreType.DMA,
        ],
        compiler_params=pltpu.CompilerParams(use_tc_tiling_on_sc=False),
    )
    def k(idx_hbm, table_hbm, out_hbm, idx_v, rows_v, sem):
        wid = lax.axis_index("s") * _NUM_CORES + lax.axis_index("c")
        base = wid * b_per_w

        def body(i, carry):
            off = base + i * _CHUNK
            pltpu.sync_copy(idx_hbm.at[pl.ds(off, _CHUNK)], idx_v)
            pltpu.async_copy(table_hbm.at[idx_v], rows_v, sem).wait()
            pltpu.sync_copy(rows_v, out_hbm.at[pl.ds(off, _CHUNK)])
            return carry

        lax.fori_loop(0, nchunks, body, 0)

    return k(idx_flat, weight)


def kernel(token_ids, weight):
    idx = token_ids.reshape(-1).astype(jnp.int32)
    out = _sc_gather(idx, weight)
    return out.reshape(token_ids.shape + (weight.shape[1],))


# speedup vs baseline: 1.1133x; 1.0171x over previous
import functools

import jax
import jax.numpy as jnp
from jax import lax
from jax.experimental import pallas as pl
from jax.experimental.pallas import tpu as pltpu
from jax.experimental.pallas import tpu_sc as plsc

_NUM_CORES = 2
_NUM_SUBCORES = 16
_NUM_WORKERS = _NUM_CORES * _NUM_SUBCORES
_CHUNK = 400
_NBUF = 4


def _sc_gather(idx3, weight):
    nw, nchunks, chunk = idx3.shape
    d = weight.shape[1]
    b_per_w = nchunks * chunk
    b_total = nw * b_per_w
    ngroups = nchunks // _NBUF
    mesh = plsc.VectorSubcoreMesh(core_axis_name="c", subcore_axis_name="s")

    @functools.partial(
        pl.kernel,
        mesh=mesh,
        out_type=jax.ShapeDtypeStruct((b_total, d), jnp.float32),
        scratch_types=[
            pltpu.VMEM((nchunks, chunk), jnp.int32),
            pltpu.VMEM((_NBUF, chunk, d), jnp.float32),
            pltpu.SemaphoreType.DMA((_NBUF,)),
            pltpu.SemaphoreType.DMA((_NBUF,)),
        ],
        compiler_params=pltpu.CompilerParams(use_tc_tiling_on_sc=False),
    )
    def k(idx_hbm, table_hbm, out_hbm, idx_v, rows_v, sem_g, sem_s):
        wid = lax.axis_index("s") * _NUM_CORES + lax.axis_index("c")
        base = wid * b_per_w
        pltpu.sync_copy(idx_hbm.at[wid], idx_v)

        def start_gather(i, b):
            pltpu.async_copy(table_hbm.at[idx_v.at[i]], rows_v.at[b], sem_g.at[b])

        def wait_gather(i, b):
            pltpu.make_async_copy(
                table_hbm.at[idx_v.at[i]], rows_v.at[b], sem_g.at[b]
            ).wait()

        def start_store(i, b):
            pltpu.async_copy(
                rows_v.at[b], out_hbm.at[pl.ds(base + i * chunk, chunk)], sem_s.at[b]
            )

        def wait_store(i, b):
            pltpu.make_async_copy(
                rows_v.at[b], out_hbm.at[pl.ds(base + i * chunk, chunk)], sem_s.at[b]
            ).wait()

        start_gather(0, 0)
        for b in range(1, _NBUF):
            start_gather(b, b)
            wait_gather(b - 1, b - 1)
            start_store(b - 1, b - 1)

        def group(g, carry):
            for b in range(_NBUF):
                i = g * _NBUF + b
                wait_store(i - _NBUF, b)
                start_gather(i, b)
                prev_b = (b - 1) % _NBUF
                wait_gather(i - 1, prev_b)
                start_store(i - 1, prev_b)
            return carry

        pl.loop(1, ngroups)(lambda g: group(g, None))

        last = nchunks - 1
        wait_gather(last, last % _NBUF)
        start_store(last, last % _NBUF)
        for b in range(_NBUF):
            wait_store(nchunks - _NBUF + b, (nchunks - _NBUF + b) % _NBUF)

    return k(idx3, weight)


def kernel(token_ids, weight):
    d = weight.shape[1]
    b_total = token_ids.shape[0] * token_ids.shape[1]
    b_per_w = b_total // _NUM_WORKERS
    nchunks = b_per_w // _CHUNK
    idx3 = token_ids.reshape(_NUM_WORKERS, nchunks, _CHUNK).astype(jnp.int32)
    out = _sc_gather(idx3, weight)
    return out.reshape(token_ids.shape + (d,))


# speedup vs baseline: 1.5010x; 1.3482x over previous
import functools

import jax
import jax.numpy as jnp
from jax import lax
from jax.experimental import pallas as pl
from jax.experimental.pallas import tpu as pltpu
from jax.experimental.pallas import tpu_sc as plsc

_NUM_CORES = 2
_NUM_SUBCORES = 16
_NUM_WORKERS = _NUM_CORES * _NUM_SUBCORES
_LANES = 16


def _sc_gather_t(tok_t, weight):
    hist, batch = tok_t.shape
    d = weight.shape[1]
    bw = batch // _NUM_WORKERS
    mesh = plsc.VectorSubcoreMesh(core_axis_name="c", subcore_axis_name="s")

    @functools.partial(
        pl.kernel,
        mesh=mesh,
        out_type=jax.ShapeDtypeStruct((hist, d, batch), jnp.float32),
        scratch_types=[
            pltpu.VMEM((hist, bw), jnp.int32),
            pltpu.VMEM((2, bw, d), jnp.float32),
            pltpu.VMEM((2, d, bw), jnp.float32),
            pltpu.SemaphoreType.DMA((2,)),
            pltpu.SemaphoreType.DMA((2,)),
        ],
        compiler_params=pltpu.CompilerParams(
            use_tc_tiling_on_sc=False, needs_layout_passes=False
        ),
    )
    def k(tok_hbm, table_hbm, out_hbm, idx_v, rows_v, trans_v, sem_g, sem_s):
        wid = lax.axis_index("s") * _NUM_CORES + lax.axis_index("c")
        b0 = wid * bw
        pltpu.sync_copy(tok_hbm.at[:, pl.ds(b0, bw)], idx_v)

        def start_gather(h, b):
            pltpu.async_copy(table_hbm.at[idx_v.at[h]], rows_v.at[b], sem_g.at[b])

        def wait_gather(h, b):
            pltpu.make_async_copy(
                table_hbm.at[idx_v.at[h]], rows_v.at[b], sem_g.at[b]
            ).wait()

        def transpose(b):
            rows = rows_v.at[b]
            trans = trans_v.at[b]

            def col_block(j0, carry):
                row_idx = j0 + jnp.arange(_LANES, dtype=jnp.int32)
                for e in range(d):
                    col_idx = jnp.full((_LANES,), e, dtype=jnp.int32)
                    vals = plsc.load_gather(rows, [row_idx, col_idx])
                    trans[e, pl.ds(j0, _LANES)] = vals
                return carry

            pl.loop(0, bw, step=_LANES)(lambda j0: col_block(j0, None))

        def fire_store(h, b):
            pltpu.async_copy(
                trans_v.at[b], out_hbm.at[h, :, pl.ds(b0, bw)], sem_s.at[b]
            )

        def wait_store(h, b):
            pltpu.make_async_copy(
                trans_v.at[b], out_hbm.at[h, :, pl.ds(b0, bw)], sem_s.at[b]
            ).wait()

        start_gather(0, 0)
        wait_gather(0, 0)
        start_gather(1, 1)
        transpose(0)
        fire_store(0, 0)
        wait_gather(1, 1)
        start_gather(2, 0)
        transpose(1)
        fire_store(1, 1)

        def body(h, b):
            wait_gather(h, b)
            start_gather(h + 1, 1 - b)
            wait_store(h - 2, b)
            transpose(b)
            fire_store(h, b)

        def pair(g, carry):
            body(2 * g, 0)
            body(2 * g + 1, 1)
            return carry

        pl.loop(1, (hist - 2) // 2)(lambda g: pair(g, None))

        body(hist - 2, (hist - 2) % 2)

        hl = hist - 1
        bl = hl % 2
        wait_gather(hl, bl)
        wait_store(hl - 2, bl)
        transpose(bl)
        fire_store(hl, bl)
        wait_store(hl - 1, 1 - bl)
        wait_store(hl, bl)

    return k(tok_t, weight)


def kernel(token_ids, weight):
    tok_t = token_ids.T.astype(jnp.int32)
    out_t = _sc_gather_t(tok_t, weight)
    return out_t.transpose(2, 0, 1)


# speedup vs baseline: 2.1888x; 1.4583x over previous
import functools

import jax
import jax.numpy as jnp
from jax import lax
from jax.experimental import pallas as pl
from jax.experimental.pallas import tpu as pltpu
from jax.experimental.pallas import tpu_sc as plsc

_NUM_CORES = 2
_NUM_SUBCORES = 16
_NUM_WORKERS = _NUM_CORES * _NUM_SUBCORES
_LANES = 16


def _sc_gather_t(tok_t, weight):
    hist, batch = tok_t.shape
    d = weight.shape[1]
    bw = batch // _NUM_WORKERS
    tstride = bw + 1
    mesh = plsc.VectorSubcoreMesh(core_axis_name="c", subcore_axis_name="s")

    @functools.partial(
        pl.kernel,
        mesh=mesh,
        out_type=jax.ShapeDtypeStruct((hist, d, batch), jnp.float32),
        scratch_types=[
            pltpu.VMEM((hist, bw), jnp.int32),
            pltpu.VMEM((2, bw, d), jnp.float32),
            pltpu.VMEM((2, d, tstride), jnp.float32),
            pltpu.SemaphoreType.DMA((2,)),
            pltpu.SemaphoreType.DMA((2,)),
        ],
        compiler_params=pltpu.CompilerParams(
            use_tc_tiling_on_sc=False, needs_layout_passes=False
        ),
    )
    def k(tok_hbm, table_hbm, out_hbm, idx_v, rows_v, trans_v, sem_g, sem_s):
        wid = lax.axis_index("s") * _NUM_CORES + lax.axis_index("c")
        b0 = wid * bw
        pltpu.sync_copy(tok_hbm.at[:, pl.ds(b0, bw)], idx_v)

        def start_gather(h, b):
            pltpu.async_copy(table_hbm.at[idx_v.at[h]], rows_v.at[b], sem_g.at[b])

        def wait_gather(h, b):
            pltpu.make_async_copy(
                table_hbm.at[idx_v.at[h]], rows_v.at[b], sem_g.at[b]
            ).wait()

        iota = jnp.arange(_LANES, dtype=jnp.int32)

        def transpose(b):
            rows = rows_v.at[b]
            trans = trans_v.at[b]

            def one_row(j, carry):
                jvec = jnp.full((_LANES,), j, dtype=jnp.int32)
                for e0 in range(d // _LANES):
                    cols = e0 * _LANES + iota
                    vals = plsc.load_gather(rows, [jvec, cols])
                    plsc.store_scatter(trans, [cols, jvec], vals)
                return carry

            pl.loop(0, bw)(lambda j: one_row(j, None))

        def fire_stores(h, b):
            pltpu.async_copy(
                trans_v.at[b, :, pl.ds(0, bw)],
                out_hbm.at[h, :, pl.ds(b0, bw)],
                sem_s.at[b],
            )

        def wait_stores(h, b):
            pltpu.make_async_copy(
                trans_v.at[b, :, pl.ds(0, bw)],
                out_hbm.at[h, :, pl.ds(b0, bw)],
                sem_s.at[b],
            ).wait()

        start_gather(0, 0)
        wait_gather(0, 0)
        start_gather(1, 1)
        transpose(0)
        fire_stores(0, 0)
        wait_gather(1, 1)
        start_gather(2, 0)
        transpose(1)
        fire_stores(1, 1)

        def body(h, b):
            wait_gather(h, b)
            start_gather(h + 1, 1 - b)
            wait_stores(h - 2, b)
            transpose(b)
            fire_stores(h, b)

        def pair(g, carry):
            body(2 * g, 0)
            body(2 * g + 1, 1)
            return carry

        pl.loop(1, (hist - 2) // 2)(lambda g: pair(g, None))

        body(hist - 2, (hist - 2) % 2)

        hl = hist - 1
        bl = hl % 2
        wait_gather(hl, bl)
        wait_stores(hl - 2, bl)
        transpose(bl)
        fire_stores(hl, bl)
        wait_stores(hl - 1, 1 - bl)
        wait_stores(hl, bl)

    return k(tok_t, weight)


def kernel(token_ids, weight):
    tok_t = token_ids.T.astype(jnp.int32)
    out_t = _sc_gather_t(tok_t, weight)
    return out_t.transpose(2, 0, 1)


# speedup vs baseline: 2.6358x; 1.2042x over previous
import functools

import jax
import jax.numpy as jnp
from jax import lax
from jax.experimental import pallas as pl
from jax.experimental.pallas import tpu as pltpu
from jax.experimental.pallas import tpu_sc as plsc

_NUM_CORES = 2
_NUM_SUBCORES = 16
_NUM_WORKERS = _NUM_CORES * _NUM_SUBCORES
_LANES = 16


def _sc_gather_t(tok_t, weight):
    hist, batch = tok_t.shape
    d = weight.shape[1]
    bw = batch // _NUM_WORKERS
    tstride = bw + 1
    mesh = plsc.VectorSubcoreMesh(core_axis_name="c", subcore_axis_name="s")

    @functools.partial(
        pl.kernel,
        mesh=mesh,
        out_type=jax.ShapeDtypeStruct((hist, d, batch), jnp.float32),
        scratch_types=[
            pltpu.VMEM((hist, bw), jnp.int32),
            pltpu.VMEM((2, bw, d), jnp.float32),
            pltpu.VMEM((2, d, tstride), jnp.float32),
            pltpu.SemaphoreType.DMA((2,)),
            pltpu.SemaphoreType.DMA((2,)),
        ],
        compiler_params=pltpu.CompilerParams(
            use_tc_tiling_on_sc=False, needs_layout_passes=False
        ),
    )
    def k(tok_hbm, table_hbm, out_hbm, idx_v, rows_v, trans_v, sem_g, sem_s):
        wid = lax.axis_index("s") * _NUM_CORES + lax.axis_index("c")
        b0 = wid * bw
        pltpu.sync_copy(tok_hbm.at[:, pl.ds(b0, bw)], idx_v)

        def start_gather(h, b):
            pltpu.async_copy(table_hbm.at[idx_v.at[h]], rows_v.at[b], sem_g.at[b])

        def wait_gather(h, b):
            pltpu.make_async_copy(
                table_hbm.at[idx_v.at[h]], rows_v.at[b], sem_g.at[b]
            ).wait()

        iota = jnp.arange(_LANES, dtype=jnp.int32)

        def transpose(b):
            rows = rows_v.at[b]
            trans = trans_v.at[b]

            def one_row(j, carry):
                jvec = jnp.full((_LANES,), j, dtype=jnp.int32)
                for e0 in range(d // _LANES):
                    cols = e0 * _LANES + iota
                    vals = plsc.load_gather(rows, [jvec, cols])
                    plsc.store_scatter(trans, [cols, jvec], vals)
                return carry

            plsc.parallel_loop(0, bw, unroll=8)(lambda j: one_row(j, None))

        def fire_stores(h, b):
            pltpu.async_copy(
                trans_v.at[b, :, pl.ds(0, bw)],
                out_hbm.at[h, :, pl.ds(b0, bw)],
                sem_s.at[b],
            )

        def wait_stores(h, b):
            pltpu.make_async_copy(
                trans_v.at[b, :, pl.ds(0, bw)],
                out_hbm.at[h, :, pl.ds(b0, bw)],
                sem_s.at[b],
            ).wait()

        start_gather(0, 0)
        wait_gather(0, 0)
        start_gather(1, 1)
        transpose(0)
        fire_stores(0, 0)
        wait_gather(1, 1)
        start_gather(2, 0)
        transpose(1)
        fire_stores(1, 1)

        def body(h, b):
            wait_gather(h, b)
            start_gather(h + 1, 1 - b)
            wait_stores(h - 2, b)
            transpose(b)
            fire_stores(h, b)

        def pair(g, carry):
            body(2 * g, 0)
            body(2 * g + 1, 1)
            return carry

        pl.loop(1, (hist - 2) // 2)(lambda g: pair(g, None))

        body(hist - 2, (hist - 2) % 2)

        hl = hist - 1
        bl = hl % 2
        wait_gather(hl, bl)
        wait_stores(hl - 2, bl)
        transpose(bl)
        fire_stores(hl, bl)
        wait_stores(hl - 1, 1 - bl)
        wait_stores(hl, bl)

    return k(tok_t, weight)


def kernel(token_ids, weight):
    tok_t = token_ids.T.astype(jnp.int32)
    out_t = _sc_gather_t(tok_t, weight)
    return out_t.transpose(2, 0, 1)
